# baseline (speedup 1.0000x reference)
import jax
import jax.numpy as jnp
from jax import lax
from jax.experimental import pallas as pl
from jax.experimental.pallas import tpu as pltpu
from jax.experimental.pallas import tpu_sc as plsc

B, F, V, D = 4096, 26, 100000, 128
DENSE, H = 13, 1024
KE = F * D
PAIRS = F // 2
NS = PAIRS * B

_NC, _NS = 2, 16
NW = _NC * _NS
PER_W = NS // NW
CHUNK = 104
N_CH = PER_W // CHUNK
NBUF = 8

BM = 512


def _gather_body(idx_hbm, table_hbm, oute_hbm, outo_hbm,
                 idxe_v, idxo_v, *bufsems):
    wid = lax.axis_index("s") * _NC + lax.axis_index("c")
    base = wid * PER_W
    bufs = bufsems[:NBUF]
    gsems = bufsems[NBUF:2 * NBUF]
    wsems = bufsems[2 * NBUF:]

    pltpu.sync_copy(idx_hbm.at[0, wid], idxe_v)
    pltpu.sync_copy(idx_hbm.at[1, wid], idxo_v)

    for idx_v, out_hbm in ((idxe_v, oute_hbm), (idxo_v, outo_hbm)):

        def out_at(c):
            return out_hbm.at[pl.ds(pl.multiple_of(base + c * CHUNK, 8), CHUNK)]

        AHEAD = NBUF // 2
        for c0 in range(AHEAD):
            pltpu.async_copy(table_hbm.at[idx_v.at[c0]], bufs[c0], gsems[c0])

        def body(i, carry):
            for b in range(NBUF):
                c = i * NBUF + b
                sp = (b + AHEAD) % NBUF

                @pl.when(c >= AHEAD)
                def _():
                    pltpu.make_async_copy(bufs[sp], out_at(c - AHEAD),
                                          wsems[sp]).wait()

                @pl.when(c + AHEAD < N_CH)
                def _():
                    pltpu.async_copy(
                        table_hbm.at[idx_v.at[jnp.minimum(c + AHEAD,
                                                          N_CH - 1)]],
                        bufs[sp], gsems[sp])

                pltpu.make_async_copy(table_hbm.at[idx_v.at[c]],
                                      bufs[b], gsems[b]).wait()
                pltpu.async_copy(bufs[b], out_at(c), wsems[b])
            return carry

        lax.fori_loop(0, N_CH // NBUF, body, 0)
        for c0 in range(N_CH - AHEAD, N_CH):
            pltpu.make_async_copy(bufs[c0 % NBUF], out_at(c0),
                                  wsems[c0 % NBUF]).wait()


def _sc_gather(idxeo, table):
    mesh = plsc.VectorSubcoreMesh(core_axis_name="c", subcore_axis_name="s")
    f = pl.kernel(
        _gather_body,
        out_type=(jax.ShapeDtypeStruct((NS, D), jnp.float32),
                  jax.ShapeDtypeStruct((NS, D), jnp.float32)),
        mesh=mesh,
        scratch_types=(
            [pltpu.VMEM((N_CH, CHUNK), jnp.int32)] * 2
            + [pltpu.VMEM((CHUNK, D), jnp.float32)] * NBUF
            + [pltpu.SemaphoreType.DMA] * (2 * NBUF)
        ),
    )
    return f(idxeo, table)


def _towers_body(fce_ref, fco_ref, dn_ref, w1c_ref, w1v_ref,
                 b1c_ref, b1v_ref, w2c_ref, w2v_ref, b2c_ref, b2v_ref,
                 octr_ref, ocvr_ref):
    xs = []
    for g in range(PAIRS):
        xe = fce_ref[g].astype(jnp.bfloat16)
        xo = fco_ref[g].astype(jnp.bfloat16)
        xs.append(jnp.concatenate([xe, xo], axis=1))
    xd = dn_ref[...].astype(jnp.bfloat16)
    accs = []
    for w1 in (w1c_ref, w1v_ref):
        acc = None
        for g in range(PAIRS):
            wg = w1[pl.ds(g * 2 * D, 2 * D), :].astype(jnp.bfloat16)
            d = jnp.dot(xs[g], wg, preferred_element_type=jnp.float32)
            acc = d if acc is None else acc + d
        wd = w1[pl.ds(KE, DENSE), :].astype(jnp.bfloat16)
        acc = acc + jnp.dot(xd, wd, preferred_element_type=jnp.float32)
        accs.append(acc)
    for acc, b1, w2, b2, oref in (
        (accs[0], b1c_ref, w2c_ref, b2c_ref, octr_ref),
        (accs[1], b1v_ref, w2v_ref, b2v_ref, ocvr_ref),
    ):
        h = jnp.maximum(acc + b1[...], 0.0)
        logit = jnp.sum(h * w2[...], axis=1) + b2[0]
        oref[...] = 1.0 / (1.0 + jnp.exp(-logit))


def _towers(fce, fco, dense, w1c, w1v, b1c, b1v, w2c, w2v, b2c, b2v):
    nb = B // BM
    rep = lambda b: (0, 0)
    rep1 = lambda b: (0,)
    return pl.pallas_call(
        _towers_body,
        grid=(nb,),
        in_specs=[
            pl.BlockSpec((PAIRS, BM, D), lambda b: (0, b, 0)),
            pl.BlockSpec((PAIRS, BM, D), lambda b: (0, b, 0)),
            pl.BlockSpec((BM, DENSE), lambda b: (b, 0)),
            pl.BlockSpec((KE + DENSE, H), rep),
            pl.BlockSpec((KE + DENSE, H), rep),
            pl.BlockSpec((H,), rep1),
            pl.BlockSpec((H,), rep1),
            pl.BlockSpec((H,), rep1),
            pl.BlockSpec((H,), rep1),
            pl.BlockSpec(memory_space=pltpu.SMEM),
            pl.BlockSpec(memory_space=pltpu.SMEM),
        ],
        out_specs=[
            pl.BlockSpec((BM,), lambda b: (b,)),
            pl.BlockSpec((BM,), lambda b: (b,)),
        ],
        out_shape=[
            jax.ShapeDtypeStruct((B,), jnp.float32),
            jax.ShapeDtypeStruct((B,), jnp.float32),
        ],
    )(fce, fco, dense, w1c, w1v, b1c, b1v, w2c, w2v, b2c, b2v)


def kernel(cat_fea_list, dense_features, table,
           W1_ctr, b1_ctr, W2_ctr, b2_ctr,
           W1_cvr, b1_cvr, W2_cvr, b2_cvr):
    idxeo = cat_fea_list.reshape(B, PAIRS, 2).transpose(2, 1, 0)
    idxeo = idxeo.reshape(2, NW, N_CH, CHUNK)
    embe, embo = _sc_gather(idxeo, table)
    fce = embe.reshape(PAIRS, B, D)
    fco = embo.reshape(PAIRS, B, D)
    octr, ocvr = _towers(fce, fco, dense_features, W1_ctr, W1_cvr,
                         b1_ctr, b1_cvr,
                         W2_ctr.reshape(H), W2_cvr.reshape(H),
                         b2_ctr, b2_cvr)
    return octr, ocvr


# speedup vs baseline: 1.1918x; 1.1918x over previous
import jax
import jax.numpy as jnp
from jax import lax
from jax.experimental import pallas as pl
from jax.experimental.pallas import tpu as pltpu
from jax.experimental.pallas import tpu_sc as plsc

B, F, V, D = 4096, 26, 100000, 128
DENSE, H = 13, 1024
KE = F * D
PAIRS = F // 2
NS = PAIRS * B

_NC, _NS = 2, 16
NW = _NC * _NS
PER_W = NS // NW
CHUNK = 104
N_CH = PER_W // CHUNK
NBUF = 8

BM = 512


def _gather_body(idx_hbm, table_hbm, oute_hbm, outo_hbm,
                 idxe_v, idxo_v, *bufsems):
    wid = lax.axis_index("s") * _NC + lax.axis_index("c")
    base = wid * PER_W
    bufs = bufsems[:NBUF]
    gsems = bufsems[NBUF:2 * NBUF]
    wsems = bufsems[2 * NBUF:]

    pltpu.sync_copy(idx_hbm.at[0, wid], idxe_v)
    pltpu.sync_copy(idx_hbm.at[1, wid], idxo_v)

    for idx_v, out_hbm in ((idxe_v, oute_hbm), (idxo_v, outo_hbm)):

        def out_at(c):
            return out_hbm.at[pl.ds(pl.multiple_of(base + c * CHUNK, 8), CHUNK)]

        AHEAD = NBUF // 2
        for c0 in range(AHEAD):
            pltpu.async_copy(table_hbm.at[idx_v.at[c0]], bufs[c0], gsems[c0])

        def body(i, carry):
            for b in range(NBUF):
                c = i * NBUF + b
                sp = (b + AHEAD) % NBUF

                @pl.when(c >= AHEAD)
                def _():
                    pltpu.make_async_copy(bufs[sp], out_at(c - AHEAD),
                                          wsems[sp]).wait()

                @pl.when(c + AHEAD < N_CH)
                def _():
                    pltpu.async_copy(
                        table_hbm.at[idx_v.at[jnp.minimum(c + AHEAD,
                                                          N_CH - 1)]],
                        bufs[sp], gsems[sp])

                pltpu.make_async_copy(table_hbm.at[idx_v.at[c]],
                                      bufs[b], gsems[b]).wait()
                pltpu.async_copy(bufs[b], out_at(c), wsems[b])
            return carry

        lax.fori_loop(0, N_CH // NBUF, body, 0)
        for c0 in range(N_CH - AHEAD, N_CH):
            pltpu.make_async_copy(bufs[c0 % NBUF], out_at(c0),
                                  wsems[c0 % NBUF]).wait()


def _sc_gather(idxeo, table):
    mesh = plsc.VectorSubcoreMesh(core_axis_name="c", subcore_axis_name="s")
    f = pl.kernel(
        _gather_body,
        out_type=(jax.ShapeDtypeStruct((NS, D), jnp.float32),
                  jax.ShapeDtypeStruct((NS, D), jnp.float32)),
        mesh=mesh,
        scratch_types=(
            [pltpu.VMEM((N_CH, CHUNK), jnp.int32)] * 2
            + [pltpu.VMEM((CHUNK, D), jnp.float32)] * NBUF
            + [pltpu.SemaphoreType.DMA] * (2 * NBUF)
        ),
    )
    return f(idxeo, table)


def _towers_body(fce_ref, fco_ref, dn_ref, w1c_ref, w1v_ref,
                 b1c_ref, b1v_ref, w2c_ref, w2v_ref, b2c_ref, b2v_ref,
                 octr_ref, ocvr_ref):
    SX = 64.0
    SW = 64.0
    f8 = jnp.float8_e4m3fn
    xs = []
    for g in range(PAIRS):
        xe = (fce_ref[g] * SX).astype(f8)
        xo = (fco_ref[g] * SX).astype(f8)
        xs.append(jnp.concatenate([xe, xo], axis=1))
    xd = dn_ref[...].astype(jnp.bfloat16)
    accs = []
    for w1 in (w1c_ref, w1v_ref):
        acc = None
        for g in range(PAIRS):
            wg = (w1[pl.ds(g * 2 * D, 2 * D), :] * SW).astype(f8)
            d = jnp.dot(xs[g], wg, preferred_element_type=jnp.float32)
            acc = d if acc is None else acc + d
        acc = acc * (1.0 / (SX * SW))
        wd = w1[pl.ds(KE, DENSE), :].astype(jnp.bfloat16)
        acc = acc + jnp.dot(xd, wd, preferred_element_type=jnp.float32)
        accs.append(acc)
    for acc, b1, w2, b2, oref in (
        (accs[0], b1c_ref, w2c_ref, b2c_ref, octr_ref),
        (accs[1], b1v_ref, w2v_ref, b2v_ref, ocvr_ref),
    ):
        h = jnp.maximum(acc + b1[...], 0.0)
        logit = jnp.sum(h * w2[...], axis=1) + b2[0]
        oref[...] = 1.0 / (1.0 + jnp.exp(-logit))


def _towers(fce, fco, dense, w1c, w1v, b1c, b1v, w2c, w2v, b2c, b2v):
    nb = B // BM
    rep = lambda b: (0, 0)
    rep1 = lambda b: (0,)
    return pl.pallas_call(
        _towers_body,
        grid=(nb,),
        in_specs=[
            pl.BlockSpec((PAIRS, BM, D), lambda b: (0, b, 0)),
            pl.BlockSpec((PAIRS, BM, D), lambda b: (0, b, 0)),
            pl.BlockSpec((BM, DENSE), lambda b: (b, 0)),
            pl.BlockSpec((KE + DENSE, H), rep),
            pl.BlockSpec((KE + DENSE, H), rep),
            pl.BlockSpec((H,), rep1),
            pl.BlockSpec((H,), rep1),
            pl.BlockSpec((H,), rep1),
            pl.BlockSpec((H,), rep1),
            pl.BlockSpec(memory_space=pltpu.SMEM),
            pl.BlockSpec(memory_space=pltpu.SMEM),
        ],
        out_specs=[
            pl.BlockSpec((BM,), lambda b: (b,)),
            pl.BlockSpec((BM,), lambda b: (b,)),
        ],
        out_shape=[
            jax.ShapeDtypeStruct((B,), jnp.float32),
            jax.ShapeDtypeStruct((B,), jnp.float32),
        ],
    )(fce, fco, dense, w1c, w1v, b1c, b1v, w2c, w2v, b2c, b2v)


def kernel(cat_fea_list, dense_features, table,
           W1_ctr, b1_ctr, W2_ctr, b2_ctr,
           W1_cvr, b1_cvr, W2_cvr, b2_cvr):
    idxeo = cat_fea_list.reshape(B, PAIRS, 2).transpose(2, 1, 0)
    idxeo = idxeo.reshape(2, NW, N_CH, CHUNK)
    embe, embo = _sc_gather(idxeo, table)
    fce = embe.reshape(PAIRS, B, D)
    fco = embo.reshape(PAIRS, B, D)
    octr, ocvr = _towers(fce, fco, dense_features, W1_ctr, W1_cvr,
                         b1_ctr, b1_cvr,
                         W2_ctr.reshape(H), W2_cvr.reshape(H),
                         b2_ctr, b2_cvr)
    return octr, ocvr


# speedup vs baseline: 1.2073x; 1.0130x over previous
import jax
import jax.numpy as jnp
from jax import lax
from jax.experimental import pallas as pl
from jax.experimental.pallas import tpu as pltpu
from jax.experimental.pallas import tpu_sc as plsc

B, F, V, D = 4096, 26, 100000, 128
DENSE, H = 13, 1024
KE = F * D
PAIRS = F // 2
NS = PAIRS * B

_NC, _NS = 2, 16
NW = _NC * _NS
PER_W = NS // NW
CHUNK = 104
N_CH = PER_W // CHUNK
NBUF = 8

BM = 512


def _gather_body(idx_hbm, table_hbm, oute_hbm, outo_hbm,
                 idxe_v, idxo_v, *bufsems):
    wid = lax.axis_index("s") * _NC + lax.axis_index("c")
    base = wid * PER_W
    bufs = bufsems[:NBUF]
    gsems = bufsems[NBUF:2 * NBUF]
    wsems = bufsems[2 * NBUF:]

    pltpu.sync_copy(idx_hbm.at[0, wid], idxe_v)
    pltpu.sync_copy(idx_hbm.at[1, wid], idxo_v)

    for idx_v, out_hbm in ((idxe_v, oute_hbm), (idxo_v, outo_hbm)):

        def out_at(c):
            return out_hbm.at[pl.ds(pl.multiple_of(base + c * CHUNK, 8), CHUNK)]

        AHEAD = NBUF // 2
        for c0 in range(AHEAD):
            pltpu.async_copy(table_hbm.at[idx_v.at[c0]], bufs[c0], gsems[c0])

        def body(i, carry):
            for b in range(NBUF):
                c = i * NBUF + b
                sp = (b + AHEAD) % NBUF

                @pl.when(c >= AHEAD)
                def _():
                    pltpu.make_async_copy(bufs[sp], out_at(c - AHEAD),
                                          wsems[sp]).wait()

                @pl.when(c + AHEAD < N_CH)
                def _():
                    pltpu.async_copy(
                        table_hbm.at[idx_v.at[jnp.minimum(c + AHEAD,
                                                          N_CH - 1)]],
                        bufs[sp], gsems[sp])

                pltpu.make_async_copy(table_hbm.at[idx_v.at[c]],
                                      bufs[b], gsems[b]).wait()
                pltpu.async_copy(bufs[b], out_at(c), wsems[b])
            return carry

        lax.fori_loop(0, N_CH // NBUF, body, 0)
        for c0 in range(N_CH - AHEAD, N_CH):
            pltpu.make_async_copy(bufs[c0 % NBUF], out_at(c0),
                                  wsems[c0 % NBUF]).wait()


def _sc_gather(idxeo, table):
    mesh = plsc.VectorSubcoreMesh(core_axis_name="c", subcore_axis_name="s")
    f = pl.kernel(
        _gather_body,
        out_type=(jax.ShapeDtypeStruct((NS, D), jnp.float32),
                  jax.ShapeDtypeStruct((NS, D), jnp.float32)),
        mesh=mesh,
        scratch_types=(
            [pltpu.VMEM((N_CH, CHUNK), jnp.int32)] * 2
            + [pltpu.VMEM((CHUNK, D), jnp.float32)] * NBUF
            + [pltpu.SemaphoreType.DMA] * (2 * NBUF)
        ),
    )
    return f(idxeo, table)


def _towers_body(fce_ref, fco_ref, dn_ref, w1c_ref, w1v_ref,
                 b1c_ref, b1v_ref, w2c_ref, w2v_ref, b2c_ref, b2v_ref,
                 octr_ref, ocvr_ref, w8c_ref, w8v_ref):
    SX = 64.0
    SW = 64.0
    f8 = jnp.float8_e4m3fn

    @pl.when(pl.program_id(0) == 0)
    def _():
        for w1, w8 in ((w1c_ref, w8c_ref), (w1v_ref, w8v_ref)):
            for g in range(PAIRS):
                w8[pl.ds(g * 2 * D, 2 * D), :] = (
                    w1[pl.ds(g * 2 * D, 2 * D), :] * SW).astype(f8)

    xs = []
    for g in range(PAIRS):
        xe = (fce_ref[g] * SX).astype(f8)
        xo = (fco_ref[g] * SX).astype(f8)
        xs.append(jnp.concatenate([xe, xo], axis=1))
    xd = dn_ref[...].astype(jnp.bfloat16)
    accs = []
    for w1, w8 in ((w1c_ref, w8c_ref), (w1v_ref, w8v_ref)):
        acc = None
        for g in range(PAIRS):
            wg = w8[pl.ds(g * 2 * D, 2 * D), :]
            d = jnp.dot(xs[g], wg, preferred_element_type=jnp.float32)
            acc = d if acc is None else acc + d
        acc = acc * (1.0 / (SX * SW))
        wd = w1[pl.ds(KE, DENSE), :].astype(jnp.bfloat16)
        acc = acc + jnp.dot(xd, wd, preferred_element_type=jnp.float32)
        accs.append(acc)
    for acc, b1, w2, b2, oref in (
        (accs[0], b1c_ref, w2c_ref, b2c_ref, octr_ref),
        (accs[1], b1v_ref, w2v_ref, b2v_ref, ocvr_ref),
    ):
        h = jnp.maximum(acc + b1[...], 0.0)
        logit = jnp.sum(h * w2[...], axis=1) + b2[0]
        oref[...] = 1.0 / (1.0 + jnp.exp(-logit))


def _towers(fce, fco, dense, w1c, w1v, b1c, b1v, w2c, w2v, b2c, b2v):
    nb = B // BM
    rep = lambda b: (0, 0)
    rep1 = lambda b: (0,)
    return pl.pallas_call(
        _towers_body,
        grid=(nb,),
        in_specs=[
            pl.BlockSpec((PAIRS, BM, D), lambda b: (0, b, 0)),
            pl.BlockSpec((PAIRS, BM, D), lambda b: (0, b, 0)),
            pl.BlockSpec((BM, DENSE), lambda b: (b, 0)),
            pl.BlockSpec((KE + DENSE, H), rep),
            pl.BlockSpec((KE + DENSE, H), rep),
            pl.BlockSpec((H,), rep1),
            pl.BlockSpec((H,), rep1),
            pl.BlockSpec((H,), rep1),
            pl.BlockSpec((H,), rep1),
            pl.BlockSpec(memory_space=pltpu.SMEM),
            pl.BlockSpec(memory_space=pltpu.SMEM),
        ],
        out_specs=[
            pl.BlockSpec((BM,), lambda b: (b,)),
            pl.BlockSpec((BM,), lambda b: (b,)),
        ],
        out_shape=[
            jax.ShapeDtypeStruct((B,), jnp.float32),
            jax.ShapeDtypeStruct((B,), jnp.float32),
        ],
        scratch_shapes=[
            pltpu.VMEM((KE, H), jnp.float8_e4m3fn),
            pltpu.VMEM((KE, H), jnp.float8_e4m3fn),
        ],
    )(fce, fco, dense, w1c, w1v, b1c, b1v, w2c, w2v, b2c, b2v)


def kernel(cat_fea_list, dense_features, table,
           W1_ctr, b1_ctr, W2_ctr, b2_ctr,
           W1_cvr, b1_cvr, W2_cvr, b2_cvr):
    idxeo = cat_fea_list.reshape(B, PAIRS, 2).transpose(2, 1, 0)
    idxeo = idxeo.reshape(2, NW, N_CH, CHUNK)
    embe, embo = _sc_gather(idxeo, table)
    fce = embe.reshape(PAIRS, B, D)
    fco = embo.reshape(PAIRS, B, D)
    octr, ocvr = _towers(fce, fco, dense_features, W1_ctr, W1_cvr,
                         b1_ctr, b1_cvr,
                         W2_ctr.reshape(H), W2_cvr.reshape(H),
                         b2_ctr, b2_cvr)
    return octr, ocvr


# speedup vs baseline: 1.2133x; 1.0050x over previous
import jax
import jax.numpy as jnp
from jax import lax
from jax.experimental import pallas as pl
from jax.experimental.pallas import tpu as pltpu
from jax.experimental.pallas import tpu_sc as plsc

B, F, V, D = 4096, 26, 100000, 128
DENSE, H = 13, 1024
KE = F * D
PAIRS = F // 2
NS = PAIRS * B

_NC, _NS = 2, 16
NW = _NC * _NS
PER_W = NS // NW
CHUNK = 104
N_CH = PER_W // CHUNK
NBUF = 8

BM = 512


def _gather_body(idx_hbm, table_hbm, oute_hbm, outo_hbm,
                 idxe_v, idxo_v, *bufsems):
    wid = lax.axis_index("s") * _NC + lax.axis_index("c")
    base = wid * PER_W
    bufs = bufsems[:NBUF]
    gsems = bufsems[NBUF:2 * NBUF]
    wsems = bufsems[2 * NBUF:]

    pltpu.sync_copy(idx_hbm.at[0, wid], idxe_v)
    pltpu.sync_copy(idx_hbm.at[1, wid], idxo_v)

    for idx_v, out_hbm in ((idxe_v, oute_hbm), (idxo_v, outo_hbm)):

        def out_at(c):
            return out_hbm.at[pl.ds(pl.multiple_of(base + c * CHUNK, 8), CHUNK)]

        AHEAD = NBUF // 2
        for c0 in range(AHEAD):
            pltpu.async_copy(table_hbm.at[idx_v.at[c0]], bufs[c0], gsems[c0])

        def body(i, carry):
            for b in range(NBUF):
                c = i * NBUF + b
                sp = (b + AHEAD) % NBUF

                @pl.when(c >= AHEAD)
                def _():
                    pltpu.make_async_copy(bufs[sp], out_at(c - AHEAD),
                                          wsems[sp]).wait()

                @pl.when(c + AHEAD < N_CH)
                def _():
                    pltpu.async_copy(
                        table_hbm.at[idx_v.at[jnp.minimum(c + AHEAD,
                                                          N_CH - 1)]],
                        bufs[sp], gsems[sp])

                pltpu.make_async_copy(table_hbm.at[idx_v.at[c]],
                                      bufs[b], gsems[b]).wait()
                pltpu.async_copy(bufs[b], out_at(c), wsems[b])
            return carry

        lax.fori_loop(0, N_CH // NBUF, body, 0)
        for c0 in range(N_CH - AHEAD, N_CH):
            pltpu.make_async_copy(bufs[c0 % NBUF], out_at(c0),
                                  wsems[c0 % NBUF]).wait()


def _sc_gather(idxeo, table):
    mesh = plsc.VectorSubcoreMesh(core_axis_name="c", subcore_axis_name="s")
    f = pl.kernel(
        _gather_body,
        out_type=(jax.ShapeDtypeStruct((NS, D), jnp.float32),
                  jax.ShapeDtypeStruct((NS, D), jnp.float32)),
        mesh=mesh,
        scratch_types=(
            [pltpu.VMEM((N_CH, CHUNK), jnp.int32)] * 2
            + [pltpu.VMEM((CHUNK, D), jnp.float32)] * NBUF
            + [pltpu.SemaphoreType.DMA] * (2 * NBUF)
        ),
    )
    return f(idxeo, table)


def _towers_body(fce_ref, fco_ref, dn_ref, w1c_ref, w1v_ref,
                 b1c_ref, b1v_ref, w2c_ref, w2v_ref, b2c_ref, b2v_ref,
                 octr_ref, ocvr_ref, w8c_ref, w8v_ref):
    SX = 64.0
    SW = 64.0
    f8 = jnp.float8_e4m3fn

    @pl.when(pl.program_id(0) == 0)
    def _():
        for w1, w8 in ((w1c_ref, w8c_ref), (w1v_ref, w8v_ref)):
            for g in range(PAIRS):
                w8[pl.ds(g * 2 * D, 2 * D), :] = (
                    w1[pl.ds(g * 2 * D, 2 * D), :] * SW).astype(f8)

    f8x = jnp.float8_e5m2
    xs = []
    for g in range(PAIRS):
        xe = fce_ref[g].astype(f8x)
        xo = fco_ref[g].astype(f8x)
        xs.append(jnp.concatenate([xe, xo], axis=1))
    xd = dn_ref[...].astype(jnp.bfloat16)
    accs = []
    for w1, w8 in ((w1c_ref, w8c_ref), (w1v_ref, w8v_ref)):
        acc = None
        for g in range(PAIRS):
            wg = w8[pl.ds(g * 2 * D, 2 * D), :]
            d = jnp.dot(xs[g], wg, preferred_element_type=jnp.float32)
            acc = d if acc is None else acc + d
        acc = acc * (1.0 / SW)
        wd = w1[pl.ds(KE, DENSE), :].astype(jnp.bfloat16)
        acc = acc + jnp.dot(xd, wd, preferred_element_type=jnp.float32)
        accs.append(acc)
    for acc, b1, w2, b2, oref in (
        (accs[0], b1c_ref, w2c_ref, b2c_ref, octr_ref),
        (accs[1], b1v_ref, w2v_ref, b2v_ref, ocvr_ref),
    ):
        h = jnp.maximum(acc + b1[...], 0.0)
        logit = jnp.sum(h * w2[...], axis=1) + b2[0]
        oref[...] = 1.0 / (1.0 + jnp.exp(-logit))


def _towers(fce, fco, dense, w1c, w1v, b1c, b1v, w2c, w2v, b2c, b2v):
    nb = B // BM
    rep = lambda b: (0, 0)
    rep1 = lambda b: (0,)
    return pl.pallas_call(
        _towers_body,
        grid=(nb,),
        in_specs=[
            pl.BlockSpec((PAIRS, BM, D), lambda b: (0, b, 0)),
            pl.BlockSpec((PAIRS, BM, D), lambda b: (0, b, 0)),
            pl.BlockSpec((BM, DENSE), lambda b: (b, 0)),
            pl.BlockSpec((KE + DENSE, H), rep),
            pl.BlockSpec((KE + DENSE, H), rep),
            pl.BlockSpec((H,), rep1),
            pl.BlockSpec((H,), rep1),
            pl.BlockSpec((H,), rep1),
            pl.BlockSpec((H,), rep1),
            pl.BlockSpec(memory_space=pltpu.SMEM),
            pl.BlockSpec(memory_space=pltpu.SMEM),
        ],
        out_specs=[
            pl.BlockSpec((BM,), lambda b: (b,)),
            pl.BlockSpec((BM,), lambda b: (b,)),
        ],
        out_shape=[
            jax.ShapeDtypeStruct((B,), jnp.float32),
            jax.ShapeDtypeStruct((B,), jnp.float32),
        ],
        scratch_shapes=[
            pltpu.VMEM((KE, H), jnp.float8_e4m3fn),
            pltpu.VMEM((KE, H), jnp.float8_e4m3fn),
        ],
    )(fce, fco, dense, w1c, w1v, b1c, b1v, w2c, w2v, b2c, b2v)


def kernel(cat_fea_list, dense_features, table,
           W1_ctr, b1_ctr, W2_ctr, b2_ctr,
           W1_cvr, b1_cvr, W2_cvr, b2_cvr):
    idxeo = cat_fea_list.reshape(B, PAIRS, 2).transpose(2, 1, 0)
    idxeo = idxeo.reshape(2, NW, N_CH, CHUNK)
    embe, embo = _sc_gather(idxeo, table)
    fce = embe.reshape(PAIRS, B, D)
    fco = embo.reshape(PAIRS, B, D)
    octr, ocvr = _towers(fce, fco, dense_features, W1_ctr, W1_cvr,
                         b1_ctr, b1_cvr,
                         W2_ctr.reshape(H), W2_cvr.reshape(H),
                         b2_ctr, b2_cvr)
    return octr, ocvr
